# baseline (speedup 1.0000x reference)
import functools

import jax
import jax.numpy as jnp
from jax import lax
from jax.experimental import pallas as pl
from jax.experimental.pallas import tpu as pltpu
from jax.experimental.pallas import tpu_sc as plsc

_B, _N, _C = 8, 20000, 8
_MAX_DET = 100
_NMS_THR = 0.5
_SCORE_THR = 0.01
_PAD = 112
_NCHUNK = _N // 16
_CM_PAD = 1280
_NSUPER = _CM_PAD // 16
_NEG_INF = float("-inf")
_FAR = 2.0e9


def _class_max_body(cls_ref, vs_ref, lbl_ref):
    x = cls_ref[...]
    m = jnp.max(x, axis=0)
    vs_ref[...] = jnp.where(m > jnp.float32(_SCORE_THR), m,
                            jnp.float32(_NEG_INF))
    ci = lax.broadcasted_iota(jnp.int32, x.shape, 0)
    lbl_ref[...] = jnp.min(jnp.where(x == m[None], ci, jnp.int32(_C)), axis=0)


def _class_max(cls_t):
    return pl.pallas_call(
        _class_max_body,
        out_shape=(
            jax.ShapeDtypeStruct((_B, _N), jnp.float32),
            jax.ShapeDtypeStruct((_B, _N), jnp.int32),
        ),
    )(cls_t)


_BOF = _N


def _nms_body(vs_hbm, bx_hbm, rot_hbm, trans_hbm, lbl_hbm,
              o_boxes, o_scores, o_labels, o_rot, o_trans,
              big_v, cm_v, sm_v,
              sel_x1, sel_y1, sel_x2, sel_y2, sel_area, sel_score,
              sel_lidx, lbl_v, buf_boxes, buf_rot, buf_trans, buf_lbl):
    wid = lax.axis_index("s") * 2 + lax.axis_index("c")
    iota = lax.iota(jnp.int32, 16)
    fneg = jnp.full((16,), _NEG_INF, jnp.float32)

    @pl.when(wid < _B)
    def _owner():
        b = wid
        pltpu.sync_copy(vs_hbm.at[b], big_v.at[pl.ds(0, _N)])
        pltpu.sync_copy(bx_hbm.at[b], big_v.at[pl.ds(_BOF, 4 * _N)])

        for k in range(_PAD // 16):
            o = k * 16
            sel_x1[pl.ds(o, 16)] = jnp.full((16,), _FAR, jnp.float32)
            sel_y1[pl.ds(o, 16)] = jnp.full((16,), _FAR, jnp.float32)
            sel_x2[pl.ds(o, 16)] = jnp.full((16,), _FAR, jnp.float32)
            sel_y2[pl.ds(o, 16)] = jnp.full((16,), _FAR, jnp.float32)
            sel_area[pl.ds(o, 16)] = jnp.zeros((16,), jnp.float32)
            sel_score[pl.ds(o, 16)] = jnp.full((16,), -1.0, jnp.float32)
            sel_lidx[pl.ds(o, 16)] = jnp.zeros((16,), jnp.int32)

        cm_v[pl.ds(_NCHUNK + 14, 16)] = fneg

        def cm_block(k, _):
            r0 = k * 16
            g0 = (r0 + iota) * 16
            acc = plsc.load_gather(big_v, [g0])
            for l in range(1, 16):
                acc = jnp.maximum(acc, plsc.load_gather(big_v, [g0 + l]))
            cm_v[pl.ds(r0, 16)] = acc
            return 0

        lax.fori_loop(0, _NCHUNK // 16, cm_block, 0)
        t0 = jnp.max(big_v[pl.ds((_NCHUNK - 2) * 16, 16)])
        t1 = jnp.max(big_v[pl.ds((_NCHUNK - 1) * 16, 16)])
        tail = jnp.where(iota == 0, t0, jnp.where(iota == 1, t1, fneg))
        cm_v[pl.ds(_NCHUNK - 2, 16)] = tail

        for k in range(_NSUPER // 16):
            g0 = (k * 16 + iota) * 16
            acc = plsc.load_gather(cm_v, [g0])
            for l in range(1, 16):
                acc = jnp.maximum(acc, plsc.load_gather(cm_v, [g0 + l]))
            sm_v[pl.ds(k * 16, 16)] = acc

        def cond(carry):
            cnt, done = carry
            return (cnt < _MAX_DET) & jnp.logical_not(done)

        lane0 = iota == 0

        def body(carry):
            cnt, _ = carry
            tops = [sm_v[pl.ds(k * 16, 16)] for k in range(_NSUPER // 16)]
            t = tops[0]
            for k in range(1, _NSUPER // 16):
                t = jnp.maximum(t, tops[k])
            m = jnp.max(t)
            done = m == jnp.float32(_NEG_INF)
            m_vec = jnp.full((16,), m, jnp.float32)

            big = jnp.full((16,), 9999, jnp.int32)
            s_vec = big
            for k in range(_NSUPER // 16):
                fk = plsc.all_reduce_ffs(tops[k] == m_vec)
                s_vec = jnp.minimum(
                    s_vec, jnp.where(fk < 16, fk + k * 16, big))
            cvec = plsc.load_gather(cm_v, [s_vec * 16 + iota])
            jc = plsc.all_reduce_ffs(cvec == m_vec)
            j_vec = s_vec * 16 + jc
            vvec = plsc.load_gather(big_v, [j_vec * 16 + iota])
            il = plsc.all_reduce_ffs(vvec == m_vec)
            i_vec = j_vec * 16 + il

            i4 = _BOF + i_vec * 4
            cx1 = plsc.load_gather(big_v, [i4])
            cy1 = plsc.load_gather(big_v, [i4 + 1])
            cx2 = plsc.load_gather(big_v, [i4 + 2])
            cy2 = plsc.load_gather(big_v, [i4 + 3])
            a1 = (cx2 - cx1) * (cy2 - cy1)
            supp = jnp.zeros((16,), jnp.bool_)
            for k in range(_PAD // 16):
                o = k * 16
                ix1 = jnp.maximum(cx1, sel_x1[pl.ds(o, 16)])
                iy1 = jnp.maximum(cy1, sel_y1[pl.ds(o, 16)])
                ix2 = jnp.minimum(cx2, sel_x2[pl.ds(o, 16)])
                iy2 = jnp.minimum(cy2, sel_y2[pl.ds(o, 16)])
                inter = (jnp.maximum(ix2 - ix1, 0.0)
                         * jnp.maximum(iy2 - iy1, 0.0))
                union = jnp.maximum(a1 + sel_area[pl.ds(o, 16)] - inter,
                                    jnp.float32(1e-8))
                supp = supp | (inter / union > jnp.float32(_NMS_THR))
            nsupp = plsc.all_reduce_population_count(supp)
            accept = (nsupp[0] == 0) & jnp.logical_not(done)

            cnt_vec = jnp.full((16,), cnt, jnp.int32)

            @pl.when(accept)
            def _take():
                for ref, val in ((sel_x1, cx1), (sel_y1, cy1),
                                 (sel_x2, cx2), (sel_y2, cy2),
                                 (sel_area, a1), (sel_score, m_vec)):
                    plsc.store_scatter(ref, [cnt_vec], val, mask=lane0)
                plsc.store_scatter(sel_lidx, [cnt_vec], i_vec, mask=lane0)

            @pl.when(jnp.logical_not(done))
            def _pop():
                new_v = jnp.where(iota == il, jnp.float32(_NEG_INF), vvec)
                plsc.store_scatter(big_v, [j_vec * 16 + iota], new_v)
                ncm = jnp.max(new_v)
                plsc.store_scatter(cm_v, [j_vec],
                                   jnp.full((16,), ncm, jnp.float32),
                                   mask=lane0)
                new_c = jnp.where(iota == jc, ncm, cvec)
                plsc.store_scatter(sm_v, [s_vec],
                                   jnp.full((16,), jnp.max(new_c),
                                            jnp.float32), mask=lane0)

            return (jnp.where(accept, cnt + 1, cnt), done)

        cnt, _ = lax.while_loop(cond, body, (jnp.int32(0), jnp.bool_(False)))

        pltpu.sync_copy(lbl_hbm.at[b], lbl_v)
        for k in range(_PAD // 16):
            o = k * 16
            pos = iota + o
            valid = pos < cnt
            sel_score[pl.ds(o, 16)] = jnp.where(
                valid, sel_score[pl.ds(o, 16)], jnp.float32(-1.0))
            lidx = sel_lidx[pl.ds(o, 16)]
            for c in range(4):
                vals = jnp.where(
                    valid, plsc.load_gather(big_v, [_BOF + lidx * 4 + c]),
                    jnp.float32(-1.0))
                plsc.store_scatter(buf_boxes, [pos, jnp.full((16,), c,
                                                             jnp.int32)], vals)
            lvals = jnp.where(valid, plsc.load_gather(lbl_v, [lidx]),
                              jnp.int32(-1))
            plsc.store_scatter(buf_lbl, [pos, jnp.zeros((16,), jnp.int32)],
                               lvals)
        for comp_hbm, buf in ((rot_hbm, buf_rot), (trans_hbm, buf_trans)):
            pltpu.sync_copy(comp_hbm.at[b], big_v.at[pl.ds(_BOF, 3 * _N)])
            for k in range(_PAD // 16):
                o = k * 16
                pos = iota + o
                valid = pos < cnt
                lidx = sel_lidx[pl.ds(o, 16)]
                for c in range(3):
                    vals = jnp.where(
                        valid, plsc.load_gather(big_v, [_BOF + lidx * 3 + c]),
                        jnp.float32(-1.0))
                    plsc.store_scatter(buf, [pos, jnp.full((16,), c,
                                                           jnp.int32)], vals)
        pltpu.sync_copy(sel_score, o_scores.at[b])
        pltpu.sync_copy(buf_boxes, o_boxes.at[b])
        pltpu.sync_copy(buf_rot, o_rot.at[b])
        pltpu.sync_copy(buf_trans, o_trans.at[b])
        pltpu.sync_copy(buf_lbl, o_labels.at[b])


def _nms_sc(vs2d, boxes_f, rot_f, trans_f, lbl2d):
    mesh = plsc.VectorSubcoreMesh(core_axis_name="c", subcore_axis_name="s",
                                  num_cores=2, num_subcores=16)
    f32, i32 = jnp.float32, jnp.int32
    return pl.kernel(
        _nms_body,
        out_type=(
            jax.ShapeDtypeStruct((_B, _PAD, 4), f32),
            jax.ShapeDtypeStruct((_B, _PAD), f32),
            jax.ShapeDtypeStruct((_B, _PAD, 1), i32),
            jax.ShapeDtypeStruct((_B, _PAD, 3), f32),
            jax.ShapeDtypeStruct((_B, _PAD, 3), f32),
        ),
        mesh=mesh,
        compiler_params=pltpu.CompilerParams(needs_layout_passes=False,
                                             use_tc_tiling_on_sc=False),
        scratch_types=[
            pltpu.VMEM((5 * _N,), f32),
            pltpu.VMEM((_CM_PAD,), f32),
            pltpu.VMEM((_NSUPER,), f32),
            pltpu.VMEM((_PAD,), f32), pltpu.VMEM((_PAD,), f32),
            pltpu.VMEM((_PAD,), f32), pltpu.VMEM((_PAD,), f32),
            pltpu.VMEM((_PAD,), f32), pltpu.VMEM((_PAD,), f32),
            pltpu.VMEM((_PAD,), i32),
            pltpu.VMEM((_N,), i32),
            pltpu.VMEM((_PAD, 4), f32),
            pltpu.VMEM((_PAD, 3), f32),
            pltpu.VMEM((_PAD, 3), f32),
            pltpu.VMEM((_PAD, 1), i32),
        ],
    )(vs2d, boxes_f, rot_f, trans_f, lbl2d)


def kernel(boxes, classification, rotation, translation):
    cls_t = classification.transpose(2, 0, 1)
    vs2d, lbl2d = _class_max(cls_t)
    boxes_f = boxes.reshape(_B, 4 * _N)
    rot_f = rotation.reshape(_B, 3 * _N)
    trans_f = translation.reshape(_B, 3 * _N)
    ob, osc, olb, orot, otr = _nms_sc(vs2d, boxes_f, rot_f, trans_f, lbl2d)
    return (ob[:, :_MAX_DET, :], osc[:, :_MAX_DET],
            olb[:, :_MAX_DET, 0], orot[:, :_MAX_DET, :],
            otr[:, :_MAX_DET, :])


# speedup vs baseline: 3.7951x; 3.7951x over previous
import functools

import jax
import jax.numpy as jnp
from jax import lax
from jax.experimental import pallas as pl
from jax.experimental.pallas import tpu as pltpu
from jax.experimental.pallas import tpu_sc as plsc

_B, _N, _C = 8, 20000, 8
_MAX_DET = 100
_NMS_THR = 0.5
_SCORE_THR = 0.01
_PAD = 112
_NCHUNK = _N // 16
_CM_PAD = 1280
_NSUPER = _CM_PAD // 16
_NEG_INF = float("-inf")
_FAR = 2.0e9


def _class_max_body(cls_ref, vs_ref, lbl_ref):
    x = cls_ref[...]
    m = jnp.max(x, axis=0)
    vs_ref[...] = jnp.where(m > jnp.float32(_SCORE_THR), m,
                            jnp.float32(_NEG_INF))
    ci = lax.broadcasted_iota(jnp.int32, x.shape, 0)
    lbl_ref[...] = jnp.min(jnp.where(x == m[None], ci, jnp.int32(_C)), axis=0)


def _class_max(cls_t):
    return pl.pallas_call(
        _class_max_body,
        out_shape=(
            jax.ShapeDtypeStruct((_B, _N), jnp.float32),
            jax.ShapeDtypeStruct((_B, _N), jnp.int32),
        ),
    )(cls_t)


_BOF = _N


def _nms_body(vs_hbm, bx_hbm, rot_hbm, trans_hbm, lbl_hbm,
              o_boxes, o_scores, o_labels, o_rot, o_trans,
              big_v, cm_v, sm_v,
              sel_x1, sel_y1, sel_x2, sel_y2, sel_area, sel_score,
              sel_lidx, lbl_v, buf_boxes, buf_rot, buf_trans, buf_lbl):
    wid = lax.axis_index("s") * 2 + lax.axis_index("c")
    iota = lax.iota(jnp.int32, 16)
    fneg = jnp.full((16,), _NEG_INF, jnp.float32)

    @pl.when(wid < _B)
    def _owner():
        b = wid
        pltpu.sync_copy(vs_hbm.at[b], big_v.at[pl.ds(0, _N)])
        for c in range(4):
            pltpu.sync_copy(bx_hbm.at[c, b],
                            big_v.at[pl.ds(_BOF + c * _N, _N)])

        for k in range(_PAD // 16):
            o = k * 16
            sel_x1[pl.ds(o, 16)] = jnp.full((16,), _FAR, jnp.float32)
            sel_y1[pl.ds(o, 16)] = jnp.full((16,), _FAR, jnp.float32)
            sel_x2[pl.ds(o, 16)] = jnp.full((16,), _FAR, jnp.float32)
            sel_y2[pl.ds(o, 16)] = jnp.full((16,), _FAR, jnp.float32)
            sel_area[pl.ds(o, 16)] = jnp.zeros((16,), jnp.float32)
            sel_score[pl.ds(o, 16)] = jnp.full((16,), -1.0, jnp.float32)
            sel_lidx[pl.ds(o, 16)] = jnp.zeros((16,), jnp.int32)

        cm_v[pl.ds(_NCHUNK + 14, 16)] = fneg

        def cm_block(k, _):
            r0 = k * 16
            g0 = (r0 + iota) * 16
            acc = plsc.load_gather(big_v, [g0])
            for l in range(1, 16):
                acc = jnp.maximum(acc, plsc.load_gather(big_v, [g0 + l]))
            cm_v[pl.ds(r0, 16)] = acc
            return 0

        lax.fori_loop(0, _NCHUNK // 16, cm_block, 0)
        t0 = jnp.max(big_v[pl.ds((_NCHUNK - 2) * 16, 16)])
        t1 = jnp.max(big_v[pl.ds((_NCHUNK - 1) * 16, 16)])
        tail = jnp.where(iota == 0, t0, jnp.where(iota == 1, t1, fneg))
        cm_v[pl.ds(_NCHUNK - 2, 16)] = tail

        for k in range(_NSUPER // 16):
            g0 = (k * 16 + iota) * 16
            acc = plsc.load_gather(cm_v, [g0])
            for l in range(1, 16):
                acc = jnp.maximum(acc, plsc.load_gather(cm_v, [g0 + l]))
            sm_v[pl.ds(k * 16, 16)] = acc

        def cond(carry):
            cnt, done = carry
            return (cnt < _MAX_DET) & jnp.logical_not(done)

        lane0 = iota == 0

        def body(carry):
            cnt, _ = carry
            tops = [sm_v[pl.ds(k * 16, 16)] for k in range(_NSUPER // 16)]
            t = tops[0]
            for k in range(1, _NSUPER // 16):
                t = jnp.maximum(t, tops[k])
            m = jnp.max(t)
            done = m == jnp.float32(_NEG_INF)
            m_vec = jnp.full((16,), m, jnp.float32)

            big = jnp.full((16,), 9999, jnp.int32)
            s_vec = big
            for k in range(_NSUPER // 16):
                fk = plsc.all_reduce_ffs(tops[k] == m_vec)
                s_vec = jnp.minimum(
                    s_vec, jnp.where(fk < 16, fk + k * 16, big))
            cvec = plsc.load_gather(cm_v, [s_vec * 16 + iota])
            jc = plsc.all_reduce_ffs(cvec == m_vec)
            j_vec = s_vec * 16 + jc
            vvec = plsc.load_gather(big_v, [j_vec * 16 + iota])
            il = plsc.all_reduce_ffs(vvec == m_vec)
            i_vec = j_vec * 16 + il

            ib = _BOF + i_vec
            cx1 = plsc.load_gather(big_v, [ib])
            cy1 = plsc.load_gather(big_v, [ib + _N])
            cx2 = plsc.load_gather(big_v, [ib + 2 * _N])
            cy2 = plsc.load_gather(big_v, [ib + 3 * _N])
            a1 = (cx2 - cx1) * (cy2 - cy1)
            supp = jnp.zeros((16,), jnp.bool_)
            for k in range(_PAD // 16):
                o = k * 16
                ix1 = jnp.maximum(cx1, sel_x1[pl.ds(o, 16)])
                iy1 = jnp.maximum(cy1, sel_y1[pl.ds(o, 16)])
                ix2 = jnp.minimum(cx2, sel_x2[pl.ds(o, 16)])
                iy2 = jnp.minimum(cy2, sel_y2[pl.ds(o, 16)])
                inter = (jnp.maximum(ix2 - ix1, 0.0)
                         * jnp.maximum(iy2 - iy1, 0.0))
                union = jnp.maximum(a1 + sel_area[pl.ds(o, 16)] - inter,
                                    jnp.float32(1e-8))
                supp = supp | (inter / union > jnp.float32(_NMS_THR))
            nsupp = plsc.all_reduce_population_count(supp)
            accept = (nsupp[0] == 0) & jnp.logical_not(done)

            cnt_vec = jnp.full((16,), cnt, jnp.int32)

            @pl.when(accept)
            def _take():
                for ref, val in ((sel_x1, cx1), (sel_y1, cy1),
                                 (sel_x2, cx2), (sel_y2, cy2),
                                 (sel_area, a1), (sel_score, m_vec)):
                    plsc.store_scatter(ref, [cnt_vec], val, mask=lane0)
                plsc.store_scatter(sel_lidx, [cnt_vec], i_vec, mask=lane0)

            @pl.when(jnp.logical_not(done))
            def _pop():
                new_v = jnp.where(iota == il, jnp.float32(_NEG_INF), vvec)
                plsc.store_scatter(big_v, [j_vec * 16 + iota], new_v)
                ncm = jnp.max(new_v)
                plsc.store_scatter(cm_v, [j_vec],
                                   jnp.full((16,), ncm, jnp.float32),
                                   mask=lane0)
                new_c = jnp.where(iota == jc, ncm, cvec)
                plsc.store_scatter(sm_v, [s_vec],
                                   jnp.full((16,), jnp.max(new_c),
                                            jnp.float32), mask=lane0)

            return (jnp.where(accept, cnt + 1, cnt), done)

        cnt, _ = lax.while_loop(cond, body, (jnp.int32(0), jnp.bool_(False)))

        pltpu.sync_copy(lbl_hbm.at[b], lbl_v)
        for k in range(_PAD // 16):
            o = k * 16
            pos = iota + o
            valid = pos < cnt
            sel_score[pl.ds(o, 16)] = jnp.where(
                valid, sel_score[pl.ds(o, 16)], jnp.float32(-1.0))
            lidx = sel_lidx[pl.ds(o, 16)]
            for c in range(4):
                vals = jnp.where(
                    valid, plsc.load_gather(big_v, [_BOF + c * _N + lidx]),
                    jnp.float32(-1.0))
                plsc.store_scatter(buf_boxes, [pos, jnp.full((16,), c,
                                                             jnp.int32)], vals)
            lvals = jnp.where(valid, plsc.load_gather(lbl_v, [lidx]),
                              jnp.int32(-1))
            plsc.store_scatter(buf_lbl, [pos, jnp.zeros((16,), jnp.int32)],
                               lvals)
        for comp_hbm, buf in ((rot_hbm, buf_rot), (trans_hbm, buf_trans)):
            for c in range(3):
                pltpu.sync_copy(comp_hbm.at[c, b],
                                big_v.at[pl.ds(_BOF + c * _N, _N)])
            for k in range(_PAD // 16):
                o = k * 16
                pos = iota + o
                valid = pos < cnt
                lidx = sel_lidx[pl.ds(o, 16)]
                for c in range(3):
                    vals = jnp.where(
                        valid, plsc.load_gather(big_v, [_BOF + c * _N + lidx]),
                        jnp.float32(-1.0))
                    plsc.store_scatter(buf, [pos, jnp.full((16,), c,
                                                           jnp.int32)], vals)
        pltpu.sync_copy(sel_score, o_scores.at[b])
        pltpu.sync_copy(buf_boxes, o_boxes.at[b])
        pltpu.sync_copy(buf_rot, o_rot.at[b])
        pltpu.sync_copy(buf_trans, o_trans.at[b])
        pltpu.sync_copy(buf_lbl, o_labels.at[b])


def _nms_sc(vs2d, boxes_f, rot_f, trans_f, lbl2d):
    mesh = plsc.VectorSubcoreMesh(core_axis_name="c", subcore_axis_name="s",
                                  num_cores=2, num_subcores=16)
    f32, i32 = jnp.float32, jnp.int32
    return pl.kernel(
        _nms_body,
        out_type=(
            jax.ShapeDtypeStruct((_B, _PAD, 4), f32),
            jax.ShapeDtypeStruct((_B, _PAD), f32),
            jax.ShapeDtypeStruct((_B, _PAD, 1), i32),
            jax.ShapeDtypeStruct((_B, _PAD, 3), f32),
            jax.ShapeDtypeStruct((_B, _PAD, 3), f32),
        ),
        mesh=mesh,
        compiler_params=pltpu.CompilerParams(needs_layout_passes=False,
                                             use_tc_tiling_on_sc=False),
        scratch_types=[
            pltpu.VMEM((5 * _N,), f32),
            pltpu.VMEM((_CM_PAD,), f32),
            pltpu.VMEM((_NSUPER,), f32),
            pltpu.VMEM((_PAD,), f32), pltpu.VMEM((_PAD,), f32),
            pltpu.VMEM((_PAD,), f32), pltpu.VMEM((_PAD,), f32),
            pltpu.VMEM((_PAD,), f32), pltpu.VMEM((_PAD,), f32),
            pltpu.VMEM((_PAD,), i32),
            pltpu.VMEM((_N,), i32),
            pltpu.VMEM((_PAD, 4), f32),
            pltpu.VMEM((_PAD, 3), f32),
            pltpu.VMEM((_PAD, 3), f32),
            pltpu.VMEM((_PAD, 1), i32),
        ],
    )(vs2d, boxes_f, rot_f, trans_f, lbl2d)


def kernel(boxes, classification, rotation, translation):
    cls_t = classification.transpose(2, 0, 1)
    vs2d, lbl2d = _class_max(cls_t)
    boxes_f = boxes.transpose(2, 0, 1)
    rot_f = rotation.transpose(2, 0, 1)
    trans_f = translation.transpose(2, 0, 1)
    ob, osc, olb, orot, otr = _nms_sc(vs2d, boxes_f, rot_f, trans_f, lbl2d)
    return (ob[:, :_MAX_DET, :], osc[:, :_MAX_DET],
            olb[:, :_MAX_DET, 0], orot[:, :_MAX_DET, :],
            otr[:, :_MAX_DET, :])


# speedup vs baseline: 4.3350x; 1.1423x over previous
import functools

import jax
import jax.numpy as jnp
from jax import lax
from jax.experimental import pallas as pl
from jax.experimental.pallas import tpu as pltpu
from jax.experimental.pallas import tpu_sc as plsc

_B, _N, _C = 8, 20000, 8
_MAX_DET = 100
_NMS_THR = 0.5
_SCORE_THR = 0.01
_PAD = 112
_NCHUNK = _N // 16
_CM_PAD = 1280
_NSUPER = _CM_PAD // 16
_NEG_INF = float("-inf")
_FAR = 2.0e9


def _class_max_body(cls_ref, vs_ref, lbl_ref):
    x = cls_ref[...]
    m = jnp.max(x, axis=0)
    vs_ref[...] = jnp.where(m > jnp.float32(_SCORE_THR), m,
                            jnp.float32(_NEG_INF))
    ci = lax.broadcasted_iota(jnp.int32, x.shape, 0)
    lbl_ref[...] = jnp.min(jnp.where(x == m[None], ci, jnp.int32(_C)), axis=0)


def _class_max(cls_t):
    return pl.pallas_call(
        _class_max_body,
        out_shape=(
            jax.ShapeDtypeStruct((_B, _N), jnp.float32),
            jax.ShapeDtypeStruct((_B, _N), jnp.int32),
        ),
    )(cls_t)


_BOF = _N


def _nms_body(vs_hbm, bx_hbm, rot_hbm, trans_hbm, lbl_hbm,
              o_boxes, o_scores, o_labels, o_rot, o_trans,
              big_v, cm_v, sm_v,
              sel_x1, sel_y1, sel_x2, sel_y2, sel_area, sel_score,
              sel_lidx, lbl_v, buf_boxes, buf_rot, buf_trans, buf_lbl, sem):
    wid = lax.axis_index("s") * 2 + lax.axis_index("c")
    iota = lax.iota(jnp.int32, 16)
    fneg = jnp.full((16,), _NEG_INF, jnp.float32)

    @pl.when(wid < _B)
    def _owner():
        b = wid
        cp_vs = pltpu.async_copy(vs_hbm.at[b], big_v.at[pl.ds(0, _N)], sem)
        cps_bx = [pltpu.async_copy(bx_hbm.at[c, b],
                                   big_v.at[pl.ds(_BOF + c * _N, _N)], sem)
                  for c in range(4)]
        cp_lbl = pltpu.async_copy(lbl_hbm.at[b], lbl_v, sem)

        for k in range(_PAD // 16):
            o = k * 16
            sel_x1[pl.ds(o, 16)] = jnp.full((16,), _FAR, jnp.float32)
            sel_y1[pl.ds(o, 16)] = jnp.full((16,), _FAR, jnp.float32)
            sel_x2[pl.ds(o, 16)] = jnp.full((16,), _FAR, jnp.float32)
            sel_y2[pl.ds(o, 16)] = jnp.full((16,), _FAR, jnp.float32)
            sel_area[pl.ds(o, 16)] = jnp.zeros((16,), jnp.float32)
            sel_score[pl.ds(o, 16)] = jnp.full((16,), -1.0, jnp.float32)
            sel_lidx[pl.ds(o, 16)] = jnp.zeros((16,), jnp.int32)

        cp_vs.wait()
        cm_v[pl.ds(_NCHUNK + 14, 16)] = fneg

        def cm_block(k, _):
            r0 = k * 16
            g0 = (r0 + iota) * 16
            acc = plsc.load_gather(big_v, [g0])
            for l in range(1, 16):
                acc = jnp.maximum(acc, plsc.load_gather(big_v, [g0 + l]))
            cm_v[pl.ds(r0, 16)] = acc
            return 0

        lax.fori_loop(0, _NCHUNK // 16, cm_block, 0)
        t0 = jnp.max(big_v[pl.ds((_NCHUNK - 2) * 16, 16)])
        t1 = jnp.max(big_v[pl.ds((_NCHUNK - 1) * 16, 16)])
        tail = jnp.where(iota == 0, t0, jnp.where(iota == 1, t1, fneg))
        cm_v[pl.ds(_NCHUNK - 2, 16)] = tail

        for k in range(_NSUPER // 16):
            g0 = (k * 16 + iota) * 16
            acc = plsc.load_gather(cm_v, [g0])
            for l in range(1, 16):
                acc = jnp.maximum(acc, plsc.load_gather(cm_v, [g0 + l]))
            sm_v[pl.ds(k * 16, 16)] = acc

        for cp in cps_bx:
            cp.wait()

        def cond(carry):
            cnt, done = carry
            return (cnt < _MAX_DET) & jnp.logical_not(done)

        lane0 = iota == 0

        def body(carry):
            cnt, _ = carry
            tops = [sm_v[pl.ds(k * 16, 16)] for k in range(_NSUPER // 16)]
            t = tops[0]
            for k in range(1, _NSUPER // 16):
                t = jnp.maximum(t, tops[k])
            m = jnp.max(t)
            done = m == jnp.float32(_NEG_INF)
            m_vec = jnp.full((16,), m, jnp.float32)

            big = jnp.full((16,), 9999, jnp.int32)
            s_vec = big
            for k in range(_NSUPER // 16):
                fk = plsc.all_reduce_ffs(tops[k] == m_vec)
                s_vec = jnp.minimum(
                    s_vec, jnp.where(fk < 16, fk + k * 16, big))
            cvec = plsc.load_gather(cm_v, [s_vec * 16 + iota])
            jc = plsc.all_reduce_ffs(cvec == m_vec)
            j_vec = s_vec * 16 + jc
            vvec = plsc.load_gather(big_v, [j_vec * 16 + iota])
            il = plsc.all_reduce_ffs(vvec == m_vec)
            i_vec = j_vec * 16 + il

            ib = _BOF + i_vec
            cx1 = plsc.load_gather(big_v, [ib])
            cy1 = plsc.load_gather(big_v, [ib + _N])
            cx2 = plsc.load_gather(big_v, [ib + 2 * _N])
            cy2 = plsc.load_gather(big_v, [ib + 3 * _N])
            a1 = (cx2 - cx1) * (cy2 - cy1)
            supp = jnp.zeros((16,), jnp.bool_)
            for k in range(_PAD // 16):
                o = k * 16
                ix1 = jnp.maximum(cx1, sel_x1[pl.ds(o, 16)])
                iy1 = jnp.maximum(cy1, sel_y1[pl.ds(o, 16)])
                ix2 = jnp.minimum(cx2, sel_x2[pl.ds(o, 16)])
                iy2 = jnp.minimum(cy2, sel_y2[pl.ds(o, 16)])
                inter = (jnp.maximum(ix2 - ix1, 0.0)
                         * jnp.maximum(iy2 - iy1, 0.0))
                union = jnp.maximum(a1 + sel_area[pl.ds(o, 16)] - inter,
                                    jnp.float32(1e-8))
                supp = supp | (inter / union > jnp.float32(_NMS_THR))
            nsupp = plsc.all_reduce_population_count(supp)
            accept = (nsupp[0] == 0) & jnp.logical_not(done)

            cnt_vec = jnp.full((16,), cnt, jnp.int32)

            @pl.when(accept)
            def _take():
                for ref, val in ((sel_x1, cx1), (sel_y1, cy1),
                                 (sel_x2, cx2), (sel_y2, cy2),
                                 (sel_area, a1), (sel_score, m_vec)):
                    plsc.store_scatter(ref, [cnt_vec], val, mask=lane0)
                plsc.store_scatter(sel_lidx, [cnt_vec], i_vec, mask=lane0)

            @pl.when(jnp.logical_not(done))
            def _pop():
                new_v = jnp.where(iota == il, jnp.float32(_NEG_INF), vvec)
                plsc.store_scatter(big_v, [j_vec * 16 + iota], new_v)
                ncm = jnp.max(new_v)
                plsc.store_scatter(cm_v, [j_vec],
                                   jnp.full((16,), ncm, jnp.float32),
                                   mask=lane0)
                new_c = jnp.where(iota == jc, ncm, cvec)
                plsc.store_scatter(sm_v, [s_vec],
                                   jnp.full((16,), jnp.max(new_c),
                                            jnp.float32), mask=lane0)

            return (jnp.where(accept, cnt + 1, cnt), done)

        cnt, _ = lax.while_loop(cond, body, (jnp.int32(0), jnp.bool_(False)))

        cp_lbl.wait()
        for k in range(_PAD // 16):
            o = k * 16
            pos = iota + o
            valid = pos < cnt
            sel_score[pl.ds(o, 16)] = jnp.where(
                valid, sel_score[pl.ds(o, 16)], jnp.float32(-1.0))
            lidx = sel_lidx[pl.ds(o, 16)]
            for c in range(4):
                vals = jnp.where(
                    valid, plsc.load_gather(big_v, [_BOF + c * _N + lidx]),
                    jnp.float32(-1.0))
                plsc.store_scatter(buf_boxes, [pos, jnp.full((16,), c,
                                                             jnp.int32)], vals)
            lvals = jnp.where(valid, plsc.load_gather(lbl_v, [lidx]),
                              jnp.int32(-1))
            plsc.store_scatter(buf_lbl, [pos, jnp.zeros((16,), jnp.int32)],
                               lvals)
        for comp_hbm, buf in ((rot_hbm, buf_rot), (trans_hbm, buf_trans)):
            cps = [pltpu.async_copy(comp_hbm.at[c, b],
                                    big_v.at[pl.ds(_BOF + c * _N, _N)], sem)
                   for c in range(3)]
            for cp in cps:
                cp.wait()
            for k in range(_PAD // 16):
                o = k * 16
                pos = iota + o
                valid = pos < cnt
                lidx = sel_lidx[pl.ds(o, 16)]
                for c in range(3):
                    vals = jnp.where(
                        valid, plsc.load_gather(big_v, [_BOF + c * _N + lidx]),
                        jnp.float32(-1.0))
                    plsc.store_scatter(buf, [pos, jnp.full((16,), c,
                                                           jnp.int32)], vals)
        pltpu.sync_copy(sel_score, o_scores.at[b])
        pltpu.sync_copy(buf_boxes, o_boxes.at[b])
        pltpu.sync_copy(buf_rot, o_rot.at[b])
        pltpu.sync_copy(buf_trans, o_trans.at[b])
        pltpu.sync_copy(buf_lbl, o_labels.at[b])


def _nms_sc(vs2d, boxes_f, rot_f, trans_f, lbl2d):
    mesh = plsc.VectorSubcoreMesh(core_axis_name="c", subcore_axis_name="s",
                                  num_cores=2, num_subcores=16)
    f32, i32 = jnp.float32, jnp.int32
    return pl.kernel(
        _nms_body,
        out_type=(
            jax.ShapeDtypeStruct((_B, _PAD, 4), f32),
            jax.ShapeDtypeStruct((_B, _PAD), f32),
            jax.ShapeDtypeStruct((_B, _PAD, 1), i32),
            jax.ShapeDtypeStruct((_B, _PAD, 3), f32),
            jax.ShapeDtypeStruct((_B, _PAD, 3), f32),
        ),
        mesh=mesh,
        compiler_params=pltpu.CompilerParams(needs_layout_passes=False,
                                             use_tc_tiling_on_sc=False),
        scratch_types=[
            pltpu.VMEM((5 * _N,), f32),
            pltpu.VMEM((_CM_PAD,), f32),
            pltpu.VMEM((_NSUPER,), f32),
            pltpu.VMEM((_PAD,), f32), pltpu.VMEM((_PAD,), f32),
            pltpu.VMEM((_PAD,), f32), pltpu.VMEM((_PAD,), f32),
            pltpu.VMEM((_PAD,), f32), pltpu.VMEM((_PAD,), f32),
            pltpu.VMEM((_PAD,), i32),
            pltpu.VMEM((_N,), i32),
            pltpu.VMEM((_PAD, 4), f32),
            pltpu.VMEM((_PAD, 3), f32),
            pltpu.VMEM((_PAD, 3), f32),
            pltpu.VMEM((_PAD, 1), i32),
            pltpu.SemaphoreType.DMA,
        ],
    )(vs2d, boxes_f, rot_f, trans_f, lbl2d)


def kernel(boxes, classification, rotation, translation):
    cls_t = classification.transpose(2, 0, 1)
    vs2d, lbl2d = _class_max(cls_t)
    boxes_f = boxes.transpose(2, 0, 1)
    rot_f = rotation.transpose(2, 0, 1)
    trans_f = translation.transpose(2, 0, 1)
    ob, osc, olb, orot, otr = _nms_sc(vs2d, boxes_f, rot_f, trans_f, lbl2d)
    return (ob[:, :_MAX_DET, :], osc[:, :_MAX_DET],
            olb[:, :_MAX_DET, 0], orot[:, :_MAX_DET, :],
            otr[:, :_MAX_DET, :])
